# Initial kernel scaffold; baseline (speedup 1.0000x reference)
import jax
import jax.numpy as jnp
from jax.experimental import pallas as pl


def kernel(x, edge_index, batch, W1, b1, W2, b2, W3, b3, W4, b4, W5, b5, W6, b6, Wc1, bc1, Wc2, bc2):
    raise NotImplementedError("write your pallas kernel here")


# speedup vs baseline: 20.4470x; 20.4470x over previous
import functools

import jax
import jax.numpy as jnp
import numpy as np
from jax import lax
from jax.experimental import pallas as pl
from jax.experimental.pallas import tpu as pltpu
from jax.experimental.pallas import tpu_sc as plsc

N = 10000
E = 320000
B = 128
NP = 10112
EPW = 10112
NW = 32
NCH = 79
ROWS_PER_TILE = 632
NEG = np.float32(-3.0e38)
f32 = jnp.float32
i32 = jnp.int32


def _make_mp(D):
    mesh = plsc.VectorSubcoreMesh(core_axis_name="c", subcore_axis_name="s")

    @functools.partial(
        pl.kernel,
        out_type=jax.ShapeDtypeStruct((2, NP, D), f32),
        mesh=mesh,
        scratch_types=[
            pltpu.VMEM((NCH, 128), i32),
            pltpu.VMEM((NCH, 128), i32),
            pltpu.VMEM((128, D), f32),
            pltpu.VMEM_SHARED((NP, D), f32),
            pltpu.SemaphoreType.DMA,
        ],
        compiler_params=pltpu.CompilerParams(use_tc_tiling_on_sc=False, needs_layout_passes=False),
    )
    def mp(y_hbm, src_hbm, dst_hbm, out_hbm, src_v, dst_v, rows_v, agg_sh, sem):
        c = lax.axis_index("c")
        s = lax.axis_index("s")
        wid = s * 2 + c

        pltpu.sync_copy(src_hbm.at[wid], src_v)
        pltpu.sync_copy(dst_hbm.at[wid], dst_v)

        dummy = jnp.full((16,), N, i32)

        def mask_body(i, _):
            r = i // 8
            m = (i % 8) * 16
            sv = src_v[r, pl.ds(m, 16)]
            dv = dst_v[r, pl.ds(m, 16)]
            dst_v[r, pl.ds(m, 16)] = jnp.where(sv == dv, dummy, dv)
            return 0

        lax.fori_loop(0, NCH * 8, mask_body, 0)

        zero16 = jnp.zeros((16,), f32)

        def zrow(i, _):
            for j in range(D // 16):
                rows_v[i, pl.ds(16 * j, 16)] = zero16
            return 0

        lax.fori_loop(0, 128, zrow, 0)
        base = s * ROWS_PER_TILE
        for off, sz in ((0, 128), (128, 128), (256, 128), (384, 128), (512, 120)):
            pltpu.sync_copy(rows_v.at[pl.ds(0, sz)], agg_sh.at[pl.ds(base + off, sz)])
        plsc.subcore_barrier()

        def step(j, _):
            pltpu.async_copy(y_hbm.at[src_v.at[j]], rows_v, sem).wait()
            pltpu.sync_copy(rows_v, agg_sh.at[dst_v.at[j]], add=True)
            return 0

        lax.fori_loop(0, NCH, step, 0)
        plsc.subcore_barrier()

        for off, sz in ((0, 128), (128, 128), (256, 128), (384, 128), (512, 120)):
            pltpu.sync_copy(agg_sh.at[pl.ds(base + off, sz)], rows_v.at[pl.ds(0, sz)])
            pltpu.sync_copy(rows_v.at[pl.ds(0, sz)], out_hbm.at[c, pl.ds(base + off, sz)])

    return mp


_mp32 = _make_mp(32)
_mp16 = _make_mp(16)


def _make_topk():
    mesh = plsc.VectorSubcoreMesh(core_axis_name="c", subcore_axis_name="s")

    @functools.partial(
        pl.kernel,
        out_type=jax.ShapeDtypeStruct((B, 32, 112), f32),
        mesh=mesh,
        scratch_types=[
            pltpu.VMEM((NP,), f32),
            pltpu.VMEM((B,), i32),
            pltpu.VMEM((B,), i32),
            pltpu.VMEM((32,), i32),
            pltpu.VMEM((32, 112), f32),
            pltpu.SemaphoreType.DMA,
        ],
        compiler_params=pltpu.CompilerParams(use_tc_tiling_on_sc=False, needs_layout_passes=False),
    )
    def tk(keys_hbm, st_hbm, cn_hbm, xc_hbm, out_hbm, keys_v, st_v, cn_v,
           idx_v, rows_v, sem):
        c = lax.axis_index("c")
        s = lax.axis_index("s")
        wid = s * 2 + c

        pltpu.sync_copy(keys_hbm, keys_v)
        pltpu.sync_copy(st_hbm, st_v)
        pltpu.sync_copy(cn_hbm, cn_v)

        iota16 = lax.iota(i32, 16)
        negk = jnp.full((16,), NEG, f32)
        dumv = jnp.full((16,), N, i32)

        for gi in range(4):
            g = wid * 4 + gi
            gv = jnp.zeros((16,), i32) + g
            s0 = jnp.max(plsc.load_gather(st_v, [gv]))
            cnt = jnp.max(plsc.load_gather(cn_v, [gv]))
            nch = (cnt + 15) // 16

            def step(i, carry):
                t0k, t0v, t1k, t1v = carry
                rel = i * 16 + iota16
                ids = s0 + rel
                m = rel < cnt
                ck = plsc.load_gather(keys_v, [ids])
                ck = jnp.where(m, ck, negk)
                cv = jnp.where(m, ids, dumv)
                ck, cv = plsc.sort_key_val(ck, cv, descending=True)
                rk = lax.rev(t1k, (0,))
                rv = lax.rev(t1v, (0,))
                ge = ck >= rk
                mk, mv = plsc.sort_key_val(
                    jnp.where(ge, ck, rk), jnp.where(ge, cv, rv),
                    descending=True)
                rmk = lax.rev(mk, (0,))
                rmv = lax.rev(mv, (0,))
                ge2 = t0k >= rmk
                nt0k, nt0v = plsc.sort_key_val(
                    jnp.where(ge2, t0k, rmk), jnp.where(ge2, t0v, rmv),
                    descending=True)
                nt1k, nt1v = plsc.sort_key_val(
                    jnp.where(ge2, rmk, t0k), jnp.where(ge2, rmv, t0v),
                    descending=True)
                return (nt0k, nt0v, nt1k, nt1v)

            t0k, t0v, t1k, t1v = lax.fori_loop(
                0, nch, step, (negk, dumv, negk, dumv))
            idx_v[pl.ds(0, 16)] = t0v
            idx_v[pl.ds(16, 16)] = t1v
            pltpu.async_copy(xc_hbm.at[idx_v], rows_v, sem).wait()
            pltpu.sync_copy(rows_v, out_hbm.at[g])

    return tk


_topk = _make_topk()


def _prep_body(x_ref, w1_ref, batch_ref, xw_ref, ones_ref, cn_ref, st_ref):
    xw_ref[...] = jnp.dot(x_ref[...], w1_ref[...],
                          preferred_element_type=f32)
    rows = lax.broadcasted_iota(i32, (NP, 16), 0)
    cols = lax.broadcasted_iota(i32, (NP, 16), 1)
    ones_ref[...] = jnp.where((rows < N) & (cols == 0), 1.0, 0.0).astype(f32)
    gids = lax.broadcasted_iota(i32, (B, NP), 0)
    eq = (batch_ref[...] == gids).astype(f32)
    counts = jnp.dot(eq, jnp.ones((NP, 1), f32),
                     preferred_element_type=f32)
    r = lax.broadcasted_iota(i32, (B, B), 0)
    q = lax.broadcasted_iota(i32, (B, B), 1)
    lt = (q < r).astype(f32)
    starts = jnp.dot(lt, counts, preferred_element_type=f32)
    cn_ref[...] = counts.astype(i32)
    st_ref[...] = starts.astype(i32)


_prep = pl.pallas_call(
    _prep_body,
    out_shape=[
        jax.ShapeDtypeStruct((NP, 32), f32),
        jax.ShapeDtypeStruct((NP, 16), f32),
        jax.ShapeDtypeStruct((B, 1), i32),
        jax.ShapeDtypeStruct((B, 1), i32),
    ],
)


def _scale_body(deg2_ref, xw_ref, dinv_ref, y_ref):
    deg = deg2_ref[0, :, 0:1] + deg2_ref[1, :, 0:1] + 1.0
    rows = lax.broadcasted_iota(i32, (NP, 1), 0)
    dinv = jnp.where(rows < N, lax.rsqrt(jnp.maximum(deg, 1.0)), 0.0)
    dinv_ref[...] = dinv
    y_ref[...] = dinv * xw_ref[...]


_scale = pl.pallas_call(
    _scale_body,
    out_shape=[
        jax.ShapeDtypeStruct((NP, 1), f32),
        jax.ShapeDtypeStruct((NP, 32), f32),
    ],
)


def _make_layer(DN):
    def body(agg_ref, y_ref, dinv_ref, b_ref, w_ref, x_ref, yn_ref):
        dinv = dinv_ref[...]
        pre = dinv * (agg_ref[0] + agg_ref[1] + y_ref[...]) + b_ref[...]
        rows = lax.broadcasted_iota(i32, (NP, 32), 0)
        xl = jnp.where(rows < N, jnp.tanh(pre), 0.0)
        x_ref[...] = xl
        yn_ref[...] = dinv * jnp.dot(xl, w_ref[...],
                                     preferred_element_type=f32)

    return pl.pallas_call(
        body,
        out_shape=[
            jax.ShapeDtypeStruct((NP, 32), f32),
            jax.ShapeDtypeStruct((NP, DN), f32),
        ],
    )


_layer32 = _make_layer(32)
_layer16 = _make_layer(16)


def _finish_body(agg_ref, y_ref, dinv_ref, b4_ref, x1_ref, x2_ref, x3_ref,
                 xc_ref, keys_ref):
    pre = dinv_ref[...] * (agg_ref[0] + agg_ref[1] + y_ref[...]) + b4_ref[...]
    rows = lax.broadcasted_iota(i32, (NP, 16), 0)
    x4w = jnp.where(rows < N, jnp.tanh(pre), 0.0)
    x4 = x4w[:, 0:1]
    xc_ref[...] = jnp.concatenate(
        [x1_ref[...], x2_ref[...], x3_ref[...], x4, jnp.zeros((NP, 15), f32)],
        axis=1)
    keys_ref[...] = x4


_finish = pl.pallas_call(
    _finish_body,
    out_shape=[
        jax.ShapeDtypeStruct((NP, 112), f32),
        jax.ShapeDtypeStruct((NP, 1), f32),
    ],
    compiler_params=pltpu.CompilerParams(vmem_limit_bytes=100 * 1024 * 1024),
)


def _tail_body(p_ref, m5_ref, b5_ref, se_ref, so_ref, g6_ref, b6_ref,
               wc1_ref, bc1_ref, wc2_ref, bc2_ref, out_ref):
    h5 = jnp.maximum(
        jnp.dot(p_ref[...], m5_ref[...], preferred_element_type=f32)
        + b5_ref[...], 0.0)
    he = jnp.dot(h5, se_ref[...], preferred_element_type=f32)
    ho = jnp.dot(h5, so_ref[...], preferred_element_type=f32)
    hp = jnp.maximum(he, ho)
    h6 = jnp.maximum(
        jnp.dot(hp, g6_ref[...], preferred_element_type=f32)
        + b6_ref[...], 0.0)
    hc = jnp.maximum(
        jnp.dot(h6, wc1_ref[...], preferred_element_type=f32)
        + bc1_ref[...], 0.0)
    out_ref[...] = (jnp.dot(hc, wc2_ref[...], preferred_element_type=f32)
                    + bc2_ref[...])


_tail = pl.pallas_call(
    _tail_body,
    out_shape=jax.ShapeDtypeStruct((B, 10), f32),
)


def _pool_selectors():
    se = np.zeros((480, 240), np.float32)
    so = np.zeros((480, 240), np.float32)
    for o in range(16):
        for u in range(15):
            se[o * 30 + 2 * u, o * 15 + u] = 1.0
            so[o * 30 + 2 * u + 1, o * 15 + u] = 1.0
    return jnp.asarray(se), jnp.asarray(so)


_SE_NP, _SO_NP = None, None


def kernel(x, edge_index, batch, W1, b1, W2, b2, W3, b3, W4, b4, W5, b5,
           W6, b6, Wc1, bc1, Wc2, bc2):
    x = x.astype(f32)

    xpad = jnp.concatenate([x, jnp.zeros((NP - N, 128), f32)], axis=0)
    batchp = jnp.concatenate(
        [batch.astype(i32), jnp.full((NP - N,), B, i32)]).reshape(1, NP)

    src = edge_index[0].astype(i32)
    dst = edge_index[1].astype(i32)
    pad_e = NW * EPW - E
    srcp = jnp.concatenate([src, jnp.zeros((pad_e,), i32)])
    dstp = jnp.concatenate([dst, jnp.full((pad_e,), N, i32)])
    src3 = srcp.reshape(NW, NCH, 128)
    dst3 = dstp.reshape(NW, NCH, 128)

    W4p = jnp.concatenate([W4, jnp.zeros((32, 15), f32)], axis=1)

    eyent = jnp.asarray(np.eye(32, 30, dtype=np.float32))
    w5e = jnp.concatenate([W5.T, jnp.zeros((15, 16), f32)], axis=0)
    m5 = jnp.einsum('nt,jo->njot', eyent, w5e).reshape(3584, 480)
    b5rep = jnp.repeat(b5, 30)

    se, so = _pool_selectors()

    dm = np.zeros((5, 15, 11), np.float32)
    for j in range(5):
        for t in range(11):
            dm[j, t + j, t] = 1.0
    g6 = jnp.einsum('ocj,jst->csot', W6, jnp.asarray(dm)).reshape(240, 352)
    b6rep = jnp.repeat(b6, 11)

    b4r = b4.reshape(1, 1)

    xw1, ones16, counts, starts = _prep(xpad, W1, batchp)
    deg2 = _mp16(ones16, src3, dst3)
    dinv, y1 = _scale(deg2, xw1)
    agg1 = _mp32(y1, src3, dst3)
    x1, y2 = _layer32(agg1, y1, dinv, b1, W2)
    agg2 = _mp32(y2, src3, dst3)
    x2, y3 = _layer32(agg2, y2, dinv, b2, W3)
    agg3 = _mp32(y3, src3, dst3)
    x3, y4 = _layer16(agg3, y3, dinv, b3, W4p)
    agg4 = _mp16(y4, src3, dst3)
    xc, keys = _finish(agg4, y4, dinv, b4r, x1, x2, x3)
    pooled = _topk(keys.reshape(NP), starts.reshape(B), counts.reshape(B), xc)
    out = _tail(pooled.reshape(B, 32 * 112), m5, b5rep, se, so, g6, b6rep,
                Wc1, bc1, Wc2, bc2)
    return out


# speedup vs baseline: 20.4691x; 1.0011x over previous
import functools

import jax
import jax.numpy as jnp
import numpy as np
from jax import lax
from jax.experimental import pallas as pl
from jax.experimental.pallas import tpu as pltpu
from jax.experimental.pallas import tpu_sc as plsc

N = 10000
E = 320000
B = 128
NP = 10112
EPW = 10112
NW = 32
NCH = 79
ROWS_PER_TILE = 632
NEG = np.float32(-3.0e38)
f32 = jnp.float32
i32 = jnp.int32


def _make_mp(D):
    mesh = plsc.VectorSubcoreMesh(core_axis_name="c", subcore_axis_name="s")

    @functools.partial(
        pl.kernel,
        out_type=jax.ShapeDtypeStruct((2, NP, D), f32),
        mesh=mesh,
        scratch_types=[
            pltpu.VMEM((NCH, 128), i32),
            pltpu.VMEM((NCH, 128), i32),
            pltpu.VMEM((128, D), f32),
            pltpu.VMEM_SHARED((NP, D), f32),
            pltpu.SemaphoreType.DMA,
        ],
        compiler_params=pltpu.CompilerParams(use_tc_tiling_on_sc=False, needs_layout_passes=False),
    )
    def mp(y_hbm, src_hbm, dst_hbm, out_hbm, src_v, dst_v, rows_v, agg_sh, sem):
        c = lax.axis_index("c")
        s = lax.axis_index("s")
        wid = s * 2 + c

        pltpu.sync_copy(src_hbm.at[wid], src_v)
        pltpu.sync_copy(dst_hbm.at[wid], dst_v)

        dummy = jnp.full((16,), N, i32)

        def mask_body(i, _):
            r = i // 8
            m = (i % 8) * 16
            sv = src_v[r, pl.ds(m, 16)]
            dv = dst_v[r, pl.ds(m, 16)]
            dst_v[r, pl.ds(m, 16)] = jnp.where(sv == dv, dummy, dv)
            return 0

        lax.fori_loop(0, NCH * 8, mask_body, 0)

        zero16 = jnp.zeros((16,), f32)

        def zrow(i, _):
            for j in range(D // 16):
                rows_v[i, pl.ds(16 * j, 16)] = zero16
            return 0

        lax.fori_loop(0, 128, zrow, 0)
        base = s * ROWS_PER_TILE
        for off, sz in ((0, 128), (128, 128), (256, 128), (384, 128), (512, 120)):
            pltpu.sync_copy(rows_v.at[pl.ds(0, sz)], agg_sh.at[pl.ds(base + off, sz)])
        plsc.subcore_barrier()

        def step(j, _):
            pltpu.async_copy(y_hbm.at[src_v.at[j]], rows_v, sem).wait()
            pltpu.sync_copy(rows_v, agg_sh.at[dst_v.at[j]], add=True)
            return 0

        lax.fori_loop(0, NCH, step, 0)
        plsc.subcore_barrier()

        for off, sz in ((0, 128), (128, 128), (256, 128), (384, 128), (512, 120)):
            pltpu.sync_copy(agg_sh.at[pl.ds(base + off, sz)], rows_v.at[pl.ds(0, sz)])
            pltpu.sync_copy(rows_v.at[pl.ds(0, sz)], out_hbm.at[c, pl.ds(base + off, sz)])

    return mp


_mp32 = _make_mp(32)
_mp16 = _make_mp(16)


def _make_topk():
    mesh = plsc.VectorSubcoreMesh(core_axis_name="c", subcore_axis_name="s")

    @functools.partial(
        pl.kernel,
        out_type=jax.ShapeDtypeStruct((B, 32, 112), f32),
        mesh=mesh,
        scratch_types=[
            pltpu.VMEM((NP,), f32),
            pltpu.VMEM((B,), i32),
            pltpu.VMEM((B,), i32),
            pltpu.VMEM((32,), i32),
            pltpu.VMEM((32, 112), f32),
            pltpu.SemaphoreType.DMA,
        ],
        compiler_params=pltpu.CompilerParams(use_tc_tiling_on_sc=False, needs_layout_passes=False),
    )
    def tk(keys_hbm, st_hbm, cn_hbm, xc_hbm, out_hbm, keys_v, st_v, cn_v,
           idx_v, rows_v, sem):
        c = lax.axis_index("c")
        s = lax.axis_index("s")
        wid = s * 2 + c

        pltpu.sync_copy(keys_hbm, keys_v)
        pltpu.sync_copy(st_hbm, st_v)
        pltpu.sync_copy(cn_hbm, cn_v)

        iota16 = lax.iota(i32, 16)
        negk = jnp.full((16,), NEG, f32)
        dumv = jnp.full((16,), N, i32)

        for gi in range(4):
            g = wid * 4 + gi
            gv = jnp.zeros((16,), i32) + g
            s0 = jnp.max(plsc.load_gather(st_v, [gv]))
            cnt = jnp.max(plsc.load_gather(cn_v, [gv]))
            nch = (cnt + 15) // 16

            def step(i, carry):
                t0k, t0v, t1k, t1v = carry
                rel = i * 16 + iota16
                ids = s0 + rel
                m = rel < cnt
                ck = plsc.load_gather(keys_v, [ids])
                ck = jnp.where(m, ck, negk)
                cv = jnp.where(m, ids, dumv)
                ck, cv = plsc.sort_key_val(ck, cv, descending=True)
                rk = lax.rev(t1k, (0,))
                rv = lax.rev(t1v, (0,))
                ge = ck >= rk
                mk, mv = plsc.sort_key_val(
                    jnp.where(ge, ck, rk), jnp.where(ge, cv, rv),
                    descending=True)
                rmk = lax.rev(mk, (0,))
                rmv = lax.rev(mv, (0,))
                ge2 = t0k >= rmk
                nt0k, nt0v = plsc.sort_key_val(
                    jnp.where(ge2, t0k, rmk), jnp.where(ge2, t0v, rmv),
                    descending=True)
                nt1k, nt1v = plsc.sort_key_val(
                    jnp.where(ge2, rmk, t0k), jnp.where(ge2, rmv, t0v),
                    descending=True)
                return (nt0k, nt0v, nt1k, nt1v)

            t0k, t0v, t1k, t1v = lax.fori_loop(
                0, nch, step, (negk, dumv, negk, dumv))
            idx_v[pl.ds(0, 16)] = t0v
            idx_v[pl.ds(16, 16)] = t1v
            pltpu.async_copy(xc_hbm.at[idx_v], rows_v, sem).wait()
            pltpu.sync_copy(rows_v, out_hbm.at[g])

    return tk


_topk = _make_topk()


def _prep_body(x_ref, w1_ref, batch_ref, xw_ref, ones_ref, cn_ref, st_ref):
    xw_ref[...] = jnp.dot(x_ref[...], w1_ref[...],
                          preferred_element_type=f32)
    rows = lax.broadcasted_iota(i32, (NP, 16), 0)
    cols = lax.broadcasted_iota(i32, (NP, 16), 1)
    ones_ref[...] = jnp.where((rows < N) & (cols == 0), 1.0, 0.0).astype(f32)
    gids = lax.broadcasted_iota(i32, (B, NP), 0)
    eq = (batch_ref[...] == gids).astype(f32)
    counts = jnp.dot(eq, jnp.ones((NP, 1), f32),
                     preferred_element_type=f32)
    r = lax.broadcasted_iota(i32, (B, B), 0)
    q = lax.broadcasted_iota(i32, (B, B), 1)
    lt = (q < r).astype(f32)
    starts = jnp.dot(lt, counts, preferred_element_type=f32)
    cn_ref[...] = counts.astype(i32)
    st_ref[...] = starts.astype(i32)


_prep = pl.pallas_call(
    _prep_body,
    out_shape=[
        jax.ShapeDtypeStruct((NP, 32), f32),
        jax.ShapeDtypeStruct((NP, 16), f32),
        jax.ShapeDtypeStruct((B, 1), i32),
        jax.ShapeDtypeStruct((B, 1), i32),
    ],
)


def _scale_body(deg2_ref, xw_ref, dinv_ref, y_ref):
    deg = deg2_ref[0, :, 0:1] + deg2_ref[1, :, 0:1] + 1.0
    rows = lax.broadcasted_iota(i32, (NP, 1), 0)
    dinv = jnp.where(rows < N, lax.rsqrt(jnp.maximum(deg, 1.0)), 0.0)
    dinv_ref[...] = dinv
    y_ref[...] = dinv * xw_ref[...]


_scale = pl.pallas_call(
    _scale_body,
    out_shape=[
        jax.ShapeDtypeStruct((NP, 1), f32),
        jax.ShapeDtypeStruct((NP, 32), f32),
    ],
)


def _make_layer(DN):
    def body(agg_ref, y_ref, dinv_ref, b_ref, w_ref, x_ref, yn_ref):
        dinv = dinv_ref[...]
        pre = dinv * (agg_ref[0] + agg_ref[1] + y_ref[...]) + b_ref[...]
        rows = lax.broadcasted_iota(i32, (NP, 32), 0)
        xl = jnp.where(rows < N, jnp.tanh(pre), 0.0)
        x_ref[...] = xl
        yn_ref[...] = dinv * jnp.dot(xl, w_ref[...],
                                     preferred_element_type=f32)

    return pl.pallas_call(
        body,
        out_shape=[
            jax.ShapeDtypeStruct((NP, 32), f32),
            jax.ShapeDtypeStruct((NP, DN), f32),
        ],
    )


_layer32 = _make_layer(32)
_layer16 = _make_layer(16)


def _finish_body(agg_ref, y_ref, dinv_ref, b4_ref, x1_ref, x2_ref, x3_ref,
                 xc_ref, keys_ref):
    pre = dinv_ref[...] * (agg_ref[0] + agg_ref[1] + y_ref[...]) + b4_ref[...]
    rows = lax.broadcasted_iota(i32, (NP, 16), 0)
    x4w = jnp.where(rows < N, jnp.tanh(pre), 0.0)
    x4 = x4w[:, 0:1]
    xc_ref[...] = jnp.concatenate(
        [x1_ref[...], x2_ref[...], x3_ref[...], x4, jnp.zeros((NP, 15), f32)],
        axis=1)
    keys_ref[...] = x4


_finish = pl.pallas_call(
    _finish_body,
    out_shape=[
        jax.ShapeDtypeStruct((NP, 112), f32),
        jax.ShapeDtypeStruct((NP, 1), f32),
    ],
    compiler_params=pltpu.CompilerParams(vmem_limit_bytes=100 * 1024 * 1024),
)


def _tail_body(p_ref, m5_ref, b5_ref, se_ref, so_ref, g6_ref, b6_ref,
               wc1_ref, bc1_ref, wc2_ref, bc2_ref, out_ref):
    h5 = jnp.maximum(
        jnp.dot(p_ref[...], m5_ref[...], preferred_element_type=f32)
        + b5_ref[...], 0.0)
    he = jnp.dot(h5, se_ref[...], preferred_element_type=f32)
    ho = jnp.dot(h5, so_ref[...], preferred_element_type=f32)
    hp = jnp.maximum(he, ho)
    h6 = jnp.maximum(
        jnp.dot(hp, g6_ref[...], preferred_element_type=f32)
        + b6_ref[...], 0.0)
    hc = jnp.maximum(
        jnp.dot(h6, wc1_ref[...], preferred_element_type=f32)
        + bc1_ref[...], 0.0)
    out_ref[...] = (jnp.dot(hc, wc2_ref[...], preferred_element_type=f32)
                    + bc2_ref[...])


_tail = pl.pallas_call(
    _tail_body,
    out_shape=jax.ShapeDtypeStruct((B, 10), f32),
)


def _pool_selectors():
    se = np.zeros((480, 240), np.float32)
    so = np.zeros((480, 240), np.float32)
    for o in range(16):
        for u in range(15):
            se[o * 30 + 2 * u, o * 15 + u] = 1.0
            so[o * 30 + 2 * u + 1, o * 15 + u] = 1.0
    return jnp.asarray(se), jnp.asarray(so)


def kernel(x, edge_index, batch, W1, b1, W2, b2, W3, b3, W4, b4, W5, b5,
           W6, b6, Wc1, bc1, Wc2, bc2):
    x = x.astype(f32)

    xpad = jnp.concatenate([x, jnp.zeros((NP - N, 128), f32)], axis=0)
    batchp = jnp.concatenate(
        [batch.astype(i32), jnp.full((NP - N,), B, i32)]).reshape(1, NP)

    src = edge_index[0].astype(i32)
    dst = edge_index[1].astype(i32)
    pad_e = NW * EPW - E
    srcp = jnp.concatenate([src, jnp.zeros((pad_e,), i32)])
    dstp = jnp.concatenate([dst, jnp.full((pad_e,), N, i32)])
    src3 = srcp.reshape(NW, NCH, 128)
    dst3 = dstp.reshape(NW, NCH, 128)

    W4p = jnp.concatenate([W4, jnp.zeros((32, 15), f32)], axis=1)

    eyent = jnp.asarray(np.eye(32, 30, dtype=np.float32))
    w5e = jnp.concatenate([W5.T, jnp.zeros((15, 16), f32)], axis=0)
    m5 = jnp.einsum('nt,jo->njot', eyent, w5e).reshape(3584, 480)
    b5rep = jnp.repeat(b5, 30)

    se, so = _pool_selectors()

    dm = np.zeros((5, 15, 11), np.float32)
    for j in range(5):
        for t in range(11):
            dm[j, t + j, t] = 1.0
    g6 = jnp.einsum('ocj,jst->csot', W6, jnp.asarray(dm)).reshape(240, 352)
    b6rep = jnp.repeat(b6, 11)

    b4r = b4.reshape(1, 1)

    xw1, ones16, counts, starts = _prep(xpad, W1, batchp)
    deg2 = _mp16(ones16, src3, dst3)
    dinv, y1 = _scale(deg2, xw1)
    agg1 = _mp32(y1, src3, dst3)
    x1, y2 = _layer32(agg1, y1, dinv, b1, W2)
    agg2 = _mp32(y2, src3, dst3)
    x2, y3 = _layer32(agg2, y2, dinv, b2, W3)
    agg3 = _mp32(y3, src3, dst3)
    x3, y4 = _layer16(agg3, y3, dinv, b3, W4p)
    agg4 = _mp16(y4, src3, dst3)
    xc, keys = _finish(agg4, y4, dinv, b4r, x1, x2, x3)
    pooled = _topk(keys.reshape(NP), starts.reshape(B), counts.reshape(B), xc)
    out = _tail(pooled.reshape(B, 32 * 112), m5, b5rep, se, so, g6, b6rep,
                Wc1, bc1, Wc2, bc2)
    return out


# speedup vs baseline: 26.4860x; 1.2940x over previous
import functools

import jax
import jax.numpy as jnp
import numpy as np
from jax import lax
from jax.experimental import pallas as pl
from jax.experimental.pallas import tpu as pltpu
from jax.experimental.pallas import tpu_sc as plsc

N = 10000
E = 320000
B = 128
NP = 10112
EPW = 10112
NW = 32
NCH = 79
ROWS_PER_TILE = 632
NEG = np.float32(-3.0e38)
f32 = jnp.float32
i32 = jnp.int32


def _make_mp(D):
    mesh = plsc.VectorSubcoreMesh(core_axis_name="c", subcore_axis_name="s")

    @functools.partial(
        pl.kernel,
        out_type=jax.ShapeDtypeStruct((2, NP, D), f32),
        mesh=mesh,
        scratch_types=[
            pltpu.VMEM((NCH, 128), i32),
            pltpu.VMEM((NCH, 128), i32),
            pltpu.VMEM((128, D), f32),
            pltpu.VMEM((128, D), f32),
            pltpu.VMEM_SHARED((NP, D), f32),
            pltpu.SemaphoreType.DMA,
        ],
        compiler_params=pltpu.CompilerParams(use_tc_tiling_on_sc=False, needs_layout_passes=False),
    )
    def mp(y_hbm, src_hbm, dst_hbm, out_hbm, src_v, dst_v, rows_v, rows_w, agg_sh, sem):
        c = lax.axis_index("c")
        s = lax.axis_index("s")
        wid = s * 2 + c

        pltpu.sync_copy(src_hbm.at[wid], src_v)
        pltpu.sync_copy(dst_hbm.at[wid], dst_v)

        dummy = jnp.full((16,), N, i32)

        def mask_body(i, _):
            r = i // 8
            m = (i % 8) * 16
            sv = src_v[r, pl.ds(m, 16)]
            dv = dst_v[r, pl.ds(m, 16)]
            dst_v[r, pl.ds(m, 16)] = jnp.where(sv == dv, dummy, dv)
            return 0

        lax.fori_loop(0, NCH * 8, mask_body, 0)

        zero16 = jnp.zeros((16,), f32)

        def zrow(i, _):
            for j in range(D // 16):
                rows_v[i, pl.ds(16 * j, 16)] = zero16
            return 0

        lax.fori_loop(0, 128, zrow, 0)
        base = s * ROWS_PER_TILE
        for off, sz in ((0, 128), (128, 128), (256, 128), (384, 128), (512, 120)):
            pltpu.sync_copy(rows_v.at[pl.ds(0, sz)], agg_sh.at[pl.ds(base + off, sz)])
        plsc.subcore_barrier()

        pltpu.async_copy(y_hbm.at[src_v.at[0]], rows_v, sem)

        def step(jj, _):
            j = 2 * jj

            @pl.when(j + 1 < NCH)
            def _():
                pltpu.async_copy(y_hbm.at[src_v.at[j + 1]], rows_w, sem)

            pltpu.make_async_copy(y_hbm.at[src_v.at[0]], rows_v, sem).wait()
            pltpu.sync_copy(rows_v, agg_sh.at[dst_v.at[j]], add=True)

            @pl.when(j + 2 < NCH)
            def _():
                pltpu.async_copy(y_hbm.at[src_v.at[j + 2]], rows_v, sem)

            @pl.when(j + 1 < NCH)
            def _():
                pltpu.make_async_copy(y_hbm.at[src_v.at[0]], rows_w, sem).wait()
                pltpu.sync_copy(rows_w, agg_sh.at[dst_v.at[j + 1]], add=True)

            return 0

        lax.fori_loop(0, (NCH + 1) // 2, step, 0)
        plsc.subcore_barrier()

        for off, sz in ((0, 128), (128, 128), (256, 128), (384, 128), (512, 120)):
            pltpu.sync_copy(agg_sh.at[pl.ds(base + off, sz)], rows_v.at[pl.ds(0, sz)])
            pltpu.sync_copy(rows_v.at[pl.ds(0, sz)], out_hbm.at[c, pl.ds(base + off, sz)])

    return mp


_mp32 = _make_mp(32)
_mp16 = _make_mp(16)


def _make_topk():
    mesh = plsc.VectorSubcoreMesh(core_axis_name="c", subcore_axis_name="s")

    @functools.partial(
        pl.kernel,
        out_type=jax.ShapeDtypeStruct((B, 32, 112), f32),
        mesh=mesh,
        scratch_types=[
            pltpu.VMEM((NP,), f32),
            pltpu.VMEM((B,), i32),
            pltpu.VMEM((B,), i32),
            pltpu.VMEM((32,), i32),
            pltpu.VMEM((32, 112), f32),
            pltpu.SemaphoreType.DMA,
        ],
        compiler_params=pltpu.CompilerParams(use_tc_tiling_on_sc=False, needs_layout_passes=False),
    )
    def tk(keys_hbm, st_hbm, cn_hbm, xc_hbm, out_hbm, keys_v, st_v, cn_v,
           idx_v, rows_v, sem):
        c = lax.axis_index("c")
        s = lax.axis_index("s")
        wid = s * 2 + c

        pltpu.sync_copy(keys_hbm, keys_v)
        pltpu.sync_copy(st_hbm, st_v)
        pltpu.sync_copy(cn_hbm, cn_v)

        iota16 = lax.iota(i32, 16)
        negk = jnp.full((16,), NEG, f32)
        dumv = jnp.full((16,), N, i32)

        for gi in range(4):
            g = wid * 4 + gi
            gv = jnp.zeros((16,), i32) + g
            s0 = jnp.max(plsc.load_gather(st_v, [gv]))
            cnt = jnp.max(plsc.load_gather(cn_v, [gv]))
            nch = (cnt + 15) // 16

            def step(i, carry):
                t0k, t0v, t1k, t1v = carry
                rel = i * 16 + iota16
                ids = s0 + rel
                m = rel < cnt
                ck = plsc.load_gather(keys_v, [ids])
                ck = jnp.where(m, ck, negk)
                cv = jnp.where(m, ids, dumv)
                ck, cv = plsc.sort_key_val(ck, cv, descending=True)
                rk = lax.rev(t1k, (0,))
                rv = lax.rev(t1v, (0,))
                ge = ck >= rk
                mk, mv = plsc.sort_key_val(
                    jnp.where(ge, ck, rk), jnp.where(ge, cv, rv),
                    descending=True)
                rmk = lax.rev(mk, (0,))
                rmv = lax.rev(mv, (0,))
                ge2 = t0k >= rmk
                nt0k, nt0v = plsc.sort_key_val(
                    jnp.where(ge2, t0k, rmk), jnp.where(ge2, t0v, rmv),
                    descending=True)
                nt1k, nt1v = plsc.sort_key_val(
                    jnp.where(ge2, rmk, t0k), jnp.where(ge2, rmv, t0v),
                    descending=True)
                return (nt0k, nt0v, nt1k, nt1v)

            t0k, t0v, t1k, t1v = lax.fori_loop(
                0, nch, step, (negk, dumv, negk, dumv))
            idx_v[pl.ds(0, 16)] = t0v
            idx_v[pl.ds(16, 16)] = t1v
            pltpu.async_copy(xc_hbm.at[idx_v], rows_v, sem).wait()
            pltpu.sync_copy(rows_v, out_hbm.at[g])

    return tk


_topk = _make_topk()


def _prep_body(x_ref, w1_ref, batch_ref, xw_ref, ones_ref, cn_ref, st_ref):
    xw_ref[...] = jnp.dot(x_ref[...], w1_ref[...],
                          preferred_element_type=f32)
    rows = lax.broadcasted_iota(i32, (NP, 16), 0)
    cols = lax.broadcasted_iota(i32, (NP, 16), 1)
    ones_ref[...] = jnp.where((rows < N) & (cols == 0), 1.0, 0.0).astype(f32)
    gids = lax.broadcasted_iota(i32, (B, NP), 0)
    eq = (batch_ref[...] == gids).astype(f32)
    counts = jnp.dot(eq, jnp.ones((NP, 1), f32),
                     preferred_element_type=f32)
    r = lax.broadcasted_iota(i32, (B, B), 0)
    q = lax.broadcasted_iota(i32, (B, B), 1)
    lt = (q < r).astype(f32)
    starts = jnp.dot(lt, counts, preferred_element_type=f32)
    cn_ref[...] = counts.astype(i32)
    st_ref[...] = starts.astype(i32)


_prep = pl.pallas_call(
    _prep_body,
    out_shape=[
        jax.ShapeDtypeStruct((NP, 32), f32),
        jax.ShapeDtypeStruct((NP, 16), f32),
        jax.ShapeDtypeStruct((B, 1), i32),
        jax.ShapeDtypeStruct((B, 1), i32),
    ],
)


def _scale_body(deg2_ref, xw_ref, dinv_ref, y_ref):
    deg = deg2_ref[0, :, 0:1] + deg2_ref[1, :, 0:1] + 1.0
    rows = lax.broadcasted_iota(i32, (NP, 1), 0)
    dinv = jnp.where(rows < N, lax.rsqrt(jnp.maximum(deg, 1.0)), 0.0)
    dinv_ref[...] = dinv
    y_ref[...] = dinv * xw_ref[...]


_scale = pl.pallas_call(
    _scale_body,
    out_shape=[
        jax.ShapeDtypeStruct((NP, 1), f32),
        jax.ShapeDtypeStruct((NP, 32), f32),
    ],
)


def _make_layer(DN):
    def body(agg_ref, y_ref, dinv_ref, b_ref, w_ref, x_ref, yn_ref):
        dinv = dinv_ref[...]
        pre = dinv * (agg_ref[0] + agg_ref[1] + y_ref[...]) + b_ref[...]
        rows = lax.broadcasted_iota(i32, (NP, 32), 0)
        xl = jnp.where(rows < N, jnp.tanh(pre), 0.0)
        x_ref[...] = xl
        yn_ref[...] = dinv * jnp.dot(xl, w_ref[...],
                                     preferred_element_type=f32)

    return pl.pallas_call(
        body,
        out_shape=[
            jax.ShapeDtypeStruct((NP, 32), f32),
            jax.ShapeDtypeStruct((NP, DN), f32),
        ],
    )


_layer32 = _make_layer(32)
_layer16 = _make_layer(16)


def _finish_body(agg_ref, y_ref, dinv_ref, b4_ref, x1_ref, x2_ref, x3_ref,
                 xc_ref, keys_ref):
    pre = dinv_ref[...] * (agg_ref[0] + agg_ref[1] + y_ref[...]) + b4_ref[...]
    rows = lax.broadcasted_iota(i32, (NP, 16), 0)
    x4w = jnp.where(rows < N, jnp.tanh(pre), 0.0)
    x4 = x4w[:, 0:1]
    xc_ref[...] = jnp.concatenate(
        [x1_ref[...], x2_ref[...], x3_ref[...], x4, jnp.zeros((NP, 15), f32)],
        axis=1)
    keys_ref[...] = x4


_finish = pl.pallas_call(
    _finish_body,
    out_shape=[
        jax.ShapeDtypeStruct((NP, 112), f32),
        jax.ShapeDtypeStruct((NP, 1), f32),
    ],
    compiler_params=pltpu.CompilerParams(vmem_limit_bytes=100 * 1024 * 1024),
)


def _tail_body(p_ref, m5_ref, b5_ref, se_ref, so_ref, g6_ref, b6_ref,
               wc1_ref, bc1_ref, wc2_ref, bc2_ref, out_ref):
    h5 = jnp.maximum(
        jnp.dot(p_ref[...], m5_ref[...], preferred_element_type=f32)
        + b5_ref[...], 0.0)
    he = jnp.dot(h5, se_ref[...], preferred_element_type=f32)
    ho = jnp.dot(h5, so_ref[...], preferred_element_type=f32)
    hp = jnp.maximum(he, ho)
    h6 = jnp.maximum(
        jnp.dot(hp, g6_ref[...], preferred_element_type=f32)
        + b6_ref[...], 0.0)
    hc = jnp.maximum(
        jnp.dot(h6, wc1_ref[...], preferred_element_type=f32)
        + bc1_ref[...], 0.0)
    out_ref[...] = (jnp.dot(hc, wc2_ref[...], preferred_element_type=f32)
                    + bc2_ref[...])


_tail = pl.pallas_call(
    _tail_body,
    out_shape=jax.ShapeDtypeStruct((B, 10), f32),
)


def _pool_selectors():
    se = np.zeros((480, 240), np.float32)
    so = np.zeros((480, 240), np.float32)
    for o in range(16):
        for u in range(15):
            se[o * 30 + 2 * u, o * 15 + u] = 1.0
            so[o * 30 + 2 * u + 1, o * 15 + u] = 1.0
    return jnp.asarray(se), jnp.asarray(so)


def kernel(x, edge_index, batch, W1, b1, W2, b2, W3, b3, W4, b4, W5, b5,
           W6, b6, Wc1, bc1, Wc2, bc2):
    x = x.astype(f32)

    xpad = jnp.concatenate([x, jnp.zeros((NP - N, 128), f32)], axis=0)
    batchp = jnp.concatenate(
        [batch.astype(i32), jnp.full((NP - N,), B, i32)]).reshape(1, NP)

    src = edge_index[0].astype(i32)
    dst = edge_index[1].astype(i32)
    pad_e = NW * EPW - E
    srcp = jnp.concatenate([src, jnp.zeros((pad_e,), i32)])
    dstp = jnp.concatenate([dst, jnp.full((pad_e,), N, i32)])
    src3 = srcp.reshape(NW, NCH, 128)
    dst3 = dstp.reshape(NW, NCH, 128)

    W4p = jnp.concatenate([W4, jnp.zeros((32, 15), f32)], axis=1)

    eyent = jnp.asarray(np.eye(32, 30, dtype=np.float32))
    w5e = jnp.concatenate([W5.T, jnp.zeros((15, 16), f32)], axis=0)
    m5 = jnp.einsum('nt,jo->njot', eyent, w5e).reshape(3584, 480)
    b5rep = jnp.repeat(b5, 30)

    se, so = _pool_selectors()

    dm = np.zeros((5, 15, 11), np.float32)
    for j in range(5):
        for t in range(11):
            dm[j, t + j, t] = 1.0
    g6 = jnp.einsum('ocj,jst->csot', W6, jnp.asarray(dm)).reshape(240, 352)
    b6rep = jnp.repeat(b6, 11)

    b4r = b4.reshape(1, 1)

    xw1, ones16, counts, starts = _prep(xpad, W1, batchp)
    deg2 = _mp16(ones16, src3, dst3)
    dinv, y1 = _scale(deg2, xw1)
    agg1 = _mp32(y1, src3, dst3)
    x1, y2 = _layer32(agg1, y1, dinv, b1, W2)
    agg2 = _mp32(y2, src3, dst3)
    x2, y3 = _layer32(agg2, y2, dinv, b2, W3)
    agg3 = _mp32(y3, src3, dst3)
    x3, y4 = _layer16(agg3, y3, dinv, b3, W4p)
    agg4 = _mp16(y4, src3, dst3)
    xc, keys = _finish(agg4, y4, dinv, b4r, x1, x2, x3)
    pooled = _topk(keys.reshape(NP), starts.reshape(B), counts.reshape(B), xc)
    out = _tail(pooled.reshape(B, 32 * 112), m5, b5rep, se, so, g6, b6rep,
                Wc1, bc1, Wc2, bc2)
    return out


# speedup vs baseline: 28.7675x; 1.0861x over previous
import functools

import jax
import jax.numpy as jnp
import numpy as np
from jax import lax
from jax.experimental import pallas as pl
from jax.experimental.pallas import tpu as pltpu
from jax.experimental.pallas import tpu_sc as plsc

N = 10000
E = 320000
B = 128
NP = 10112
EPW = 10112
NW = 32
NCH = 79
ROWS_PER_TILE = 632
NEG = np.float32(-3.0e38)
f32 = jnp.float32
i32 = jnp.int32


def _make_mp(D):
    mesh = plsc.VectorSubcoreMesh(core_axis_name="c", subcore_axis_name="s")

    @functools.partial(
        pl.kernel,
        out_type=jax.ShapeDtypeStruct((2, NP, D), f32),
        mesh=mesh,
        scratch_types=[
            pltpu.VMEM((NCH, 128), i32),
            pltpu.VMEM((NCH, 128), i32),
            pltpu.VMEM((4, 128, D), f32),
            pltpu.VMEM_SHARED((NP, D), f32),
            pltpu.SemaphoreType.DMA,
        ],
        compiler_params=pltpu.CompilerParams(use_tc_tiling_on_sc=False, needs_layout_passes=False),
    )
    def mp(y_hbm, src_hbm, dst_hbm, out_hbm, src_v, dst_v, ring_v, agg_sh, sem):
        c = lax.axis_index("c")
        s = lax.axis_index("s")
        wid = s * 2 + c

        pltpu.sync_copy(src_hbm.at[wid], src_v)
        pltpu.sync_copy(dst_hbm.at[wid], dst_v)

        dummy = jnp.full((16,), N, i32)

        def mask_body(i, _):
            r = i // 8
            m = (i % 8) * 16
            sv = src_v[r, pl.ds(m, 16)]
            dv = dst_v[r, pl.ds(m, 16)]
            dst_v[r, pl.ds(m, 16)] = jnp.where(sv == dv, dummy, dv)
            return 0

        lax.fori_loop(0, NCH * 8, mask_body, 0)

        zero16 = jnp.zeros((16,), f32)

        def zrow(i, _):
            for j in range(D // 16):
                ring_v[0, i, pl.ds(16 * j, 16)] = zero16
            return 0

        lax.fori_loop(0, 128, zrow, 0)
        base = s * ROWS_PER_TILE
        for off, sz in ((0, 128), (128, 128), (256, 128), (384, 128), (512, 120)):
            pltpu.sync_copy(ring_v.at[0, pl.ds(0, sz)], agg_sh.at[pl.ds(base + off, sz)])
        plsc.subcore_barrier()

        for b in range(4):
            pltpu.async_copy(y_hbm.at[src_v.at[b]], ring_v.at[b], sem)

        def step(jj, _):
            j = 4 * jj
            for b in range(4):
                pltpu.make_async_copy(y_hbm.at[src_v.at[0]], ring_v.at[b], sem).wait()
                pltpu.sync_copy(ring_v.at[b], agg_sh.at[dst_v.at[j + b]], add=True)

                @pl.when(j + b + 4 < NCH)
                def _():
                    pltpu.async_copy(y_hbm.at[src_v.at[j + b + 4]], ring_v.at[b], sem)

            return 0

        lax.fori_loop(0, NCH // 4, step, 0)
        for b in range(NCH % 4):
            pltpu.make_async_copy(y_hbm.at[src_v.at[0]], ring_v.at[b], sem).wait()
            pltpu.sync_copy(ring_v.at[b], agg_sh.at[dst_v.at[(NCH // 4) * 4 + b]], add=True)
        plsc.subcore_barrier()

        for off, sz in ((0, 128), (128, 128), (256, 128), (384, 128), (512, 120)):
            pltpu.sync_copy(agg_sh.at[pl.ds(base + off, sz)], ring_v.at[0, pl.ds(0, sz)])
            pltpu.sync_copy(ring_v.at[0, pl.ds(0, sz)], out_hbm.at[c, pl.ds(base + off, sz)])

    return mp


_mp32 = _make_mp(32)
_mp16 = _make_mp(16)


def _make_topk():
    mesh = plsc.VectorSubcoreMesh(core_axis_name="c", subcore_axis_name="s")

    @functools.partial(
        pl.kernel,
        out_type=jax.ShapeDtypeStruct((B, 32, 112), f32),
        mesh=mesh,
        scratch_types=[
            pltpu.VMEM((NP,), f32),
            pltpu.VMEM((B,), i32),
            pltpu.VMEM((B,), i32),
            pltpu.VMEM((32,), i32),
            pltpu.VMEM((32, 112), f32),
            pltpu.SemaphoreType.DMA,
        ],
        compiler_params=pltpu.CompilerParams(use_tc_tiling_on_sc=False, needs_layout_passes=False),
    )
    def tk(keys_hbm, st_hbm, cn_hbm, xc_hbm, out_hbm, keys_v, st_v, cn_v,
           idx_v, rows_v, sem):
        c = lax.axis_index("c")
        s = lax.axis_index("s")
        wid = s * 2 + c

        pltpu.sync_copy(keys_hbm, keys_v)
        pltpu.sync_copy(st_hbm, st_v)
        pltpu.sync_copy(cn_hbm, cn_v)

        iota16 = lax.iota(i32, 16)
        negk = jnp.full((16,), NEG, f32)
        dumv = jnp.full((16,), N, i32)

        for gi in range(4):
            g = wid * 4 + gi
            gv = jnp.zeros((16,), i32) + g
            s0 = jnp.max(plsc.load_gather(st_v, [gv]))
            cnt = jnp.max(plsc.load_gather(cn_v, [gv]))
            nch = (cnt + 15) // 16

            def step(i, carry):
                t0k, t0v, t1k, t1v = carry
                rel = i * 16 + iota16
                ids = s0 + rel
                m = rel < cnt
                ck = plsc.load_gather(keys_v, [ids])
                ck = jnp.where(m, ck, negk)
                cv = jnp.where(m, ids, dumv)
                ck, cv = plsc.sort_key_val(ck, cv, descending=True)
                rk = lax.rev(t1k, (0,))
                rv = lax.rev(t1v, (0,))
                ge = ck >= rk
                mk, mv = plsc.sort_key_val(
                    jnp.where(ge, ck, rk), jnp.where(ge, cv, rv),
                    descending=True)
                rmk = lax.rev(mk, (0,))
                rmv = lax.rev(mv, (0,))
                ge2 = t0k >= rmk
                nt0k, nt0v = plsc.sort_key_val(
                    jnp.where(ge2, t0k, rmk), jnp.where(ge2, t0v, rmv),
                    descending=True)
                nt1k, nt1v = plsc.sort_key_val(
                    jnp.where(ge2, rmk, t0k), jnp.where(ge2, rmv, t0v),
                    descending=True)
                return (nt0k, nt0v, nt1k, nt1v)

            t0k, t0v, t1k, t1v = lax.fori_loop(
                0, nch, step, (negk, dumv, negk, dumv))
            idx_v[pl.ds(0, 16)] = t0v
            idx_v[pl.ds(16, 16)] = t1v
            pltpu.async_copy(xc_hbm.at[idx_v], rows_v, sem).wait()
            pltpu.sync_copy(rows_v, out_hbm.at[g])

    return tk


_topk = _make_topk()


def _prep_body(x_ref, w1_ref, batch_ref, xw_ref, ones_ref, cn_ref, st_ref):
    xw_ref[...] = jnp.dot(x_ref[...], w1_ref[...],
                          preferred_element_type=f32)
    rows = lax.broadcasted_iota(i32, (NP, 16), 0)
    cols = lax.broadcasted_iota(i32, (NP, 16), 1)
    ones_ref[...] = jnp.where((rows < N) & (cols == 0), 1.0, 0.0).astype(f32)
    gids = lax.broadcasted_iota(i32, (B, NP), 0)
    eq = (batch_ref[...] == gids).astype(f32)
    counts = jnp.dot(eq, jnp.ones((NP, 1), f32),
                     preferred_element_type=f32)
    r = lax.broadcasted_iota(i32, (B, B), 0)
    q = lax.broadcasted_iota(i32, (B, B), 1)
    lt = (q < r).astype(f32)
    starts = jnp.dot(lt, counts, preferred_element_type=f32)
    cn_ref[...] = counts.astype(i32)
    st_ref[...] = starts.astype(i32)


_prep = pl.pallas_call(
    _prep_body,
    out_shape=[
        jax.ShapeDtypeStruct((NP, 32), f32),
        jax.ShapeDtypeStruct((NP, 16), f32),
        jax.ShapeDtypeStruct((B, 1), i32),
        jax.ShapeDtypeStruct((B, 1), i32),
    ],
)


def _scale_body(deg2_ref, xw_ref, dinv_ref, y_ref):
    deg = deg2_ref[0, :, 0:1] + deg2_ref[1, :, 0:1] + 1.0
    rows = lax.broadcasted_iota(i32, (NP, 1), 0)
    dinv = jnp.where(rows < N, lax.rsqrt(jnp.maximum(deg, 1.0)), 0.0)
    dinv_ref[...] = dinv
    y_ref[...] = dinv * xw_ref[...]


_scale = pl.pallas_call(
    _scale_body,
    out_shape=[
        jax.ShapeDtypeStruct((NP, 1), f32),
        jax.ShapeDtypeStruct((NP, 32), f32),
    ],
)


def _make_layer(DN):
    def body(agg_ref, y_ref, dinv_ref, b_ref, w_ref, x_ref, yn_ref):
        dinv = dinv_ref[...]
        pre = dinv * (agg_ref[0] + agg_ref[1] + y_ref[...]) + b_ref[...]
        rows = lax.broadcasted_iota(i32, (NP, 32), 0)
        xl = jnp.where(rows < N, jnp.tanh(pre), 0.0)
        x_ref[...] = xl
        yn_ref[...] = dinv * jnp.dot(xl, w_ref[...],
                                     preferred_element_type=f32)

    return pl.pallas_call(
        body,
        out_shape=[
            jax.ShapeDtypeStruct((NP, 32), f32),
            jax.ShapeDtypeStruct((NP, DN), f32),
        ],
    )


_layer32 = _make_layer(32)
_layer16 = _make_layer(16)


def _finish_body(agg_ref, y_ref, dinv_ref, b4_ref, x1_ref, x2_ref, x3_ref,
                 xc_ref, keys_ref):
    pre = dinv_ref[...] * (agg_ref[0] + agg_ref[1] + y_ref[...]) + b4_ref[...]
    rows = lax.broadcasted_iota(i32, (NP, 16), 0)
    x4w = jnp.where(rows < N, jnp.tanh(pre), 0.0)
    x4 = x4w[:, 0:1]
    xc_ref[...] = jnp.concatenate(
        [x1_ref[...], x2_ref[...], x3_ref[...], x4, jnp.zeros((NP, 15), f32)],
        axis=1)
    keys_ref[...] = x4


_finish = pl.pallas_call(
    _finish_body,
    out_shape=[
        jax.ShapeDtypeStruct((NP, 112), f32),
        jax.ShapeDtypeStruct((NP, 1), f32),
    ],
    compiler_params=pltpu.CompilerParams(vmem_limit_bytes=100 * 1024 * 1024),
)


def _tail_body(p_ref, m5_ref, b5_ref, se_ref, so_ref, g6_ref, b6_ref,
               wc1_ref, bc1_ref, wc2_ref, bc2_ref, out_ref):
    h5 = jnp.maximum(
        jnp.dot(p_ref[...], m5_ref[...], preferred_element_type=f32)
        + b5_ref[...], 0.0)
    he = jnp.dot(h5, se_ref[...], preferred_element_type=f32)
    ho = jnp.dot(h5, so_ref[...], preferred_element_type=f32)
    hp = jnp.maximum(he, ho)
    h6 = jnp.maximum(
        jnp.dot(hp, g6_ref[...], preferred_element_type=f32)
        + b6_ref[...], 0.0)
    hc = jnp.maximum(
        jnp.dot(h6, wc1_ref[...], preferred_element_type=f32)
        + bc1_ref[...], 0.0)
    out_ref[...] = (jnp.dot(hc, wc2_ref[...], preferred_element_type=f32)
                    + bc2_ref[...])


_tail = pl.pallas_call(
    _tail_body,
    out_shape=jax.ShapeDtypeStruct((B, 10), f32),
)


def _pool_selectors():
    se = np.zeros((480, 240), np.float32)
    so = np.zeros((480, 240), np.float32)
    for o in range(16):
        for u in range(15):
            se[o * 30 + 2 * u, o * 15 + u] = 1.0
            so[o * 30 + 2 * u + 1, o * 15 + u] = 1.0
    return jnp.asarray(se), jnp.asarray(so)


def kernel(x, edge_index, batch, W1, b1, W2, b2, W3, b3, W4, b4, W5, b5,
           W6, b6, Wc1, bc1, Wc2, bc2):
    x = x.astype(f32)

    xpad = jnp.concatenate([x, jnp.zeros((NP - N, 128), f32)], axis=0)
    batchp = jnp.concatenate(
        [batch.astype(i32), jnp.full((NP - N,), B, i32)]).reshape(1, NP)

    src = edge_index[0].astype(i32)
    dst = edge_index[1].astype(i32)
    pad_e = NW * EPW - E
    srcp = jnp.concatenate([src, jnp.zeros((pad_e,), i32)])
    dstp = jnp.concatenate([dst, jnp.full((pad_e,), N, i32)])
    src3 = srcp.reshape(NW, NCH, 128)
    dst3 = dstp.reshape(NW, NCH, 128)

    W4p = jnp.concatenate([W4, jnp.zeros((32, 15), f32)], axis=1)

    eyent = jnp.asarray(np.eye(32, 30, dtype=np.float32))
    w5e = jnp.concatenate([W5.T, jnp.zeros((15, 16), f32)], axis=0)
    m5 = jnp.einsum('nt,jo->njot', eyent, w5e).reshape(3584, 480)
    b5rep = jnp.repeat(b5, 30)

    se, so = _pool_selectors()

    dm = np.zeros((5, 15, 11), np.float32)
    for j in range(5):
        for t in range(11):
            dm[j, t + j, t] = 1.0
    g6 = jnp.einsum('ocj,jst->csot', W6, jnp.asarray(dm)).reshape(240, 352)
    b6rep = jnp.repeat(b6, 11)

    b4r = b4.reshape(1, 1)

    xw1, ones16, counts, starts = _prep(xpad, W1, batchp)
    deg2 = _mp16(ones16, src3, dst3)
    dinv, y1 = _scale(deg2, xw1)
    agg1 = _mp32(y1, src3, dst3)
    x1, y2 = _layer32(agg1, y1, dinv, b1, W2)
    agg2 = _mp32(y2, src3, dst3)
    x2, y3 = _layer32(agg2, y2, dinv, b2, W3)
    agg3 = _mp32(y3, src3, dst3)
    x3, y4 = _layer16(agg3, y3, dinv, b3, W4p)
    agg4 = _mp16(y4, src3, dst3)
    xc, keys = _finish(agg4, y4, dinv, b4r, x1, x2, x3)
    pooled = _topk(keys.reshape(NP), starts.reshape(B), counts.reshape(B), xc)
    out = _tail(pooled.reshape(B, 32 * 112), m5, b5rep, se, so, g6, b6rep,
                Wc1, bc1, Wc2, bc2)
    return out


# speedup vs baseline: 30.4993x; 1.0602x over previous
import functools

import jax
import jax.numpy as jnp
import numpy as np
from jax import lax
from jax.experimental import pallas as pl
from jax.experimental.pallas import tpu as pltpu
from jax.experimental.pallas import tpu_sc as plsc

N = 10000
E = 320000
B = 128
NP = 10112
EPW = 10112
NW = 32
NCH = 79
ROWS_PER_TILE = 632
NEG = np.float32(-3.0e38)
f32 = jnp.float32
i32 = jnp.int32


def _make_mp(D, const_ones=False):
    mesh = plsc.VectorSubcoreMesh(core_axis_name="c", subcore_axis_name="s")

    @functools.partial(
        pl.kernel,
        out_type=jax.ShapeDtypeStruct((2, NP, D), f32),
        mesh=mesh,
        scratch_types=[
            pltpu.VMEM((NCH, 128), i32),
            pltpu.VMEM((NCH, 128), i32),
            pltpu.VMEM((4, 128, D), f32),
            pltpu.VMEM_SHARED((NP, D), f32),
            pltpu.SemaphoreType.DMA,
        ],
        compiler_params=pltpu.CompilerParams(use_tc_tiling_on_sc=False, needs_layout_passes=False),
    )
    def mp(y_hbm, src_hbm, dst_hbm, out_hbm, src_v, dst_v, ring_v, agg_sh, sem):
        c = lax.axis_index("c")
        s = lax.axis_index("s")
        wid = s * 2 + c

        pltpu.sync_copy(src_hbm.at[wid], src_v)
        pltpu.sync_copy(dst_hbm.at[wid], dst_v)

        dummy = jnp.full((16,), N, i32)

        def mask_body(i, _):
            r = i // 8
            m = (i % 8) * 16
            sv = src_v[r, pl.ds(m, 16)]
            dv = dst_v[r, pl.ds(m, 16)]
            dst_v[r, pl.ds(m, 16)] = jnp.where(sv == dv, dummy, dv)
            return 0

        lax.fori_loop(0, NCH * 8, mask_body, 0)

        zero16 = jnp.zeros((16,), f32)

        def zrow(i, _):
            for j in range(D // 16):
                ring_v[0, i, pl.ds(16 * j, 16)] = zero16
            return 0

        lax.fori_loop(0, 128, zrow, 0)
        base = s * ROWS_PER_TILE
        for off, sz in ((0, 128), (128, 128), (256, 128), (384, 128), (512, 120)):
            pltpu.sync_copy(ring_v.at[0, pl.ds(0, sz)], agg_sh.at[pl.ds(base + off, sz)])
        plsc.subcore_barrier()

        if const_ones:
            onehot = (lax.iota(i32, 16) == 0).astype(f32)

            def orow(i, _):
                ring_v[0, i, pl.ds(0, 16)] = onehot
                for j in range(1, D // 16):
                    ring_v[0, i, pl.ds(16 * j, 16)] = zero16
                return 0

            lax.fori_loop(0, 128, orow, 0)

            def step1(j, _):
                pltpu.sync_copy(ring_v.at[0], agg_sh.at[dst_v.at[j]], add=True)
                return 0

            lax.fori_loop(0, NCH, step1, 0)
        else:
            for b in range(4):
                pltpu.async_copy(y_hbm.at[src_v.at[b]], ring_v.at[b], sem)

            def step(jj, _):
                j = 4 * jj
                for b in range(4):
                    pltpu.make_async_copy(y_hbm.at[src_v.at[0]], ring_v.at[b], sem).wait()
                    pltpu.sync_copy(ring_v.at[b], agg_sh.at[dst_v.at[j + b]], add=True)

                    @pl.when(j + b + 4 < NCH)
                    def _():
                        pltpu.async_copy(y_hbm.at[src_v.at[j + b + 4]], ring_v.at[b], sem)

                return 0

            lax.fori_loop(0, NCH // 4, step, 0)
            for b in range(NCH % 4):
                pltpu.make_async_copy(y_hbm.at[src_v.at[0]], ring_v.at[b], sem).wait()
                pltpu.sync_copy(ring_v.at[b], agg_sh.at[dst_v.at[(NCH // 4) * 4 + b]], add=True)
        plsc.subcore_barrier()

        for off, sz in ((0, 128), (128, 128), (256, 128), (384, 128), (512, 120)):
            pltpu.sync_copy(agg_sh.at[pl.ds(base + off, sz)], ring_v.at[0, pl.ds(0, sz)])
            pltpu.sync_copy(ring_v.at[0, pl.ds(0, sz)], out_hbm.at[c, pl.ds(base + off, sz)])

    return mp


_mp32 = _make_mp(32)
_mp16 = _make_mp(16)
_deg = _make_mp(16, const_ones=True)


def _make_topk():
    mesh = plsc.VectorSubcoreMesh(core_axis_name="c", subcore_axis_name="s")

    @functools.partial(
        pl.kernel,
        out_type=jax.ShapeDtypeStruct((B, 32, 112), f32),
        mesh=mesh,
        scratch_types=[
            pltpu.VMEM((NP,), f32),
            pltpu.VMEM((B,), i32),
            pltpu.VMEM((B,), i32),
            pltpu.VMEM((32,), i32),
            pltpu.VMEM((32, 112), f32),
            pltpu.SemaphoreType.DMA,
        ],
        compiler_params=pltpu.CompilerParams(use_tc_tiling_on_sc=False, needs_layout_passes=False),
    )
    def tk(keys_hbm, st_hbm, cn_hbm, xc_hbm, out_hbm, keys_v, st_v, cn_v,
           idx_v, rows_v, sem):
        c = lax.axis_index("c")
        s = lax.axis_index("s")
        wid = s * 2 + c

        pltpu.sync_copy(keys_hbm, keys_v)
        pltpu.sync_copy(st_hbm, st_v)
        pltpu.sync_copy(cn_hbm, cn_v)

        iota16 = lax.iota(i32, 16)
        negk = jnp.full((16,), NEG, f32)
        dumv = jnp.full((16,), N, i32)

        for gi in range(4):
            g = wid * 4 + gi
            gv = jnp.zeros((16,), i32) + g
            s0 = jnp.max(plsc.load_gather(st_v, [gv]))
            cnt = jnp.max(plsc.load_gather(cn_v, [gv]))
            nch = (cnt + 15) // 16

            def step(i, carry):
                t0k, t0v, t1k, t1v = carry
                rel = i * 16 + iota16
                ids = s0 + rel
                m = rel < cnt
                ck = plsc.load_gather(keys_v, [ids])
                ck = jnp.where(m, ck, negk)
                cv = jnp.where(m, ids, dumv)
                ck, cv = plsc.sort_key_val(ck, cv, descending=True)
                rk = lax.rev(t1k, (0,))
                rv = lax.rev(t1v, (0,))
                ge = ck >= rk
                mk, mv = plsc.sort_key_val(
                    jnp.where(ge, ck, rk), jnp.where(ge, cv, rv),
                    descending=True)
                rmk = lax.rev(mk, (0,))
                rmv = lax.rev(mv, (0,))
                ge2 = t0k >= rmk
                nt0k, nt0v = plsc.sort_key_val(
                    jnp.where(ge2, t0k, rmk), jnp.where(ge2, t0v, rmv),
                    descending=True)
                nt1k, nt1v = plsc.sort_key_val(
                    jnp.where(ge2, rmk, t0k), jnp.where(ge2, rmv, t0v),
                    descending=True)
                return (nt0k, nt0v, nt1k, nt1v)

            t0k, t0v, t1k, t1v = lax.fori_loop(
                0, nch, step, (negk, dumv, negk, dumv))
            idx_v[pl.ds(0, 16)] = t0v
            idx_v[pl.ds(16, 16)] = t1v
            pltpu.async_copy(xc_hbm.at[idx_v], rows_v, sem).wait()
            pltpu.sync_copy(rows_v, out_hbm.at[g])

    return tk


_topk = _make_topk()


def _prep_body(x_ref, w1_ref, batch_ref, xw_ref, ones_ref, cn_ref, st_ref):
    xw_ref[...] = jnp.dot(x_ref[...], w1_ref[...],
                          preferred_element_type=f32)
    rows = lax.broadcasted_iota(i32, (NP, 16), 0)
    cols = lax.broadcasted_iota(i32, (NP, 16), 1)
    ones_ref[...] = jnp.where((rows < N) & (cols == 0), 1.0, 0.0).astype(f32)
    gids = lax.broadcasted_iota(i32, (B, NP), 0)
    eq = (batch_ref[...] == gids).astype(f32)
    counts = jnp.dot(eq, jnp.ones((NP, 1), f32),
                     preferred_element_type=f32)
    r = lax.broadcasted_iota(i32, (B, B), 0)
    q = lax.broadcasted_iota(i32, (B, B), 1)
    lt = (q < r).astype(f32)
    starts = jnp.dot(lt, counts, preferred_element_type=f32)
    cn_ref[...] = counts.astype(i32)
    st_ref[...] = starts.astype(i32)


_prep = pl.pallas_call(
    _prep_body,
    out_shape=[
        jax.ShapeDtypeStruct((NP, 32), f32),
        jax.ShapeDtypeStruct((NP, 16), f32),
        jax.ShapeDtypeStruct((B, 1), i32),
        jax.ShapeDtypeStruct((B, 1), i32),
    ],
)


def _scale_body(deg2_ref, xw_ref, dinv_ref, y_ref):
    deg = deg2_ref[0, :, 0:1] + deg2_ref[1, :, 0:1] + 1.0
    rows = lax.broadcasted_iota(i32, (NP, 1), 0)
    dinv = jnp.where(rows < N, lax.rsqrt(jnp.maximum(deg, 1.0)), 0.0)
    dinv_ref[...] = dinv
    y_ref[...] = dinv * xw_ref[...]


_scale = pl.pallas_call(
    _scale_body,
    out_shape=[
        jax.ShapeDtypeStruct((NP, 1), f32),
        jax.ShapeDtypeStruct((NP, 32), f32),
    ],
)


def _make_layer(DN):
    def body(agg_ref, y_ref, dinv_ref, b_ref, w_ref, x_ref, yn_ref):
        dinv = dinv_ref[...]
        pre = dinv * (agg_ref[0] + agg_ref[1] + y_ref[...]) + b_ref[...]
        rows = lax.broadcasted_iota(i32, (NP, 32), 0)
        xl = jnp.where(rows < N, jnp.tanh(pre), 0.0)
        x_ref[...] = xl
        yn_ref[...] = dinv * jnp.dot(xl, w_ref[...],
                                     preferred_element_type=f32)

    return pl.pallas_call(
        body,
        out_shape=[
            jax.ShapeDtypeStruct((NP, 32), f32),
            jax.ShapeDtypeStruct((NP, DN), f32),
        ],
    )


_layer32 = _make_layer(32)
_layer16 = _make_layer(16)


def _finish_body(agg_ref, y_ref, dinv_ref, b4_ref, x1_ref, x2_ref, x3_ref,
                 xc_ref, keys_ref):
    pre = dinv_ref[...] * (agg_ref[0] + agg_ref[1] + y_ref[...]) + b4_ref[...]
    rows = lax.broadcasted_iota(i32, (NP, 16), 0)
    x4w = jnp.where(rows < N, jnp.tanh(pre), 0.0)
    x4 = x4w[:, 0:1]
    xc_ref[...] = jnp.concatenate(
        [x1_ref[...], x2_ref[...], x3_ref[...], x4, jnp.zeros((NP, 15), f32)],
        axis=1)
    keys_ref[...] = x4


_finish = pl.pallas_call(
    _finish_body,
    out_shape=[
        jax.ShapeDtypeStruct((NP, 112), f32),
        jax.ShapeDtypeStruct((NP, 1), f32),
    ],
    compiler_params=pltpu.CompilerParams(vmem_limit_bytes=100 * 1024 * 1024),
)


def _tail_body(p_ref, m5_ref, b5_ref, se_ref, so_ref, g6_ref, b6_ref,
               wc1_ref, bc1_ref, wc2_ref, bc2_ref, out_ref):
    h5 = jnp.maximum(
        jnp.dot(p_ref[...], m5_ref[...], preferred_element_type=f32)
        + b5_ref[...], 0.0)
    he = jnp.dot(h5, se_ref[...], preferred_element_type=f32)
    ho = jnp.dot(h5, so_ref[...], preferred_element_type=f32)
    hp = jnp.maximum(he, ho)
    h6 = jnp.maximum(
        jnp.dot(hp, g6_ref[...], preferred_element_type=f32)
        + b6_ref[...], 0.0)
    hc = jnp.maximum(
        jnp.dot(h6, wc1_ref[...], preferred_element_type=f32)
        + bc1_ref[...], 0.0)
    out_ref[...] = (jnp.dot(hc, wc2_ref[...], preferred_element_type=f32)
                    + bc2_ref[...])


_tail = pl.pallas_call(
    _tail_body,
    out_shape=jax.ShapeDtypeStruct((B, 10), f32),
)


def _pool_selectors():
    se = np.zeros((480, 240), np.float32)
    so = np.zeros((480, 240), np.float32)
    for o in range(16):
        for u in range(15):
            se[o * 30 + 2 * u, o * 15 + u] = 1.0
            so[o * 30 + 2 * u + 1, o * 15 + u] = 1.0
    return jnp.asarray(se), jnp.asarray(so)


def kernel(x, edge_index, batch, W1, b1, W2, b2, W3, b3, W4, b4, W5, b5,
           W6, b6, Wc1, bc1, Wc2, bc2):
    x = x.astype(f32)

    xpad = jnp.concatenate([x, jnp.zeros((NP - N, 128), f32)], axis=0)
    batchp = jnp.concatenate(
        [batch.astype(i32), jnp.full((NP - N,), B, i32)]).reshape(1, NP)

    src = edge_index[0].astype(i32)
    dst = edge_index[1].astype(i32)
    pad_e = NW * EPW - E
    srcp = jnp.concatenate([src, jnp.zeros((pad_e,), i32)])
    dstp = jnp.concatenate([dst, jnp.full((pad_e,), N, i32)])
    src3 = srcp.reshape(NW, NCH, 128)
    dst3 = dstp.reshape(NW, NCH, 128)

    W4p = jnp.concatenate([W4, jnp.zeros((32, 15), f32)], axis=1)

    eyent = jnp.asarray(np.eye(32, 30, dtype=np.float32))
    w5e = jnp.concatenate([W5.T, jnp.zeros((15, 16), f32)], axis=0)
    m5 = jnp.einsum('nt,jo->njot', eyent, w5e).reshape(3584, 480)
    b5rep = jnp.repeat(b5, 30)

    se, so = _pool_selectors()

    dm = np.zeros((5, 15, 11), np.float32)
    for j in range(5):
        for t in range(11):
            dm[j, t + j, t] = 1.0
    g6 = jnp.einsum('ocj,jst->csot', W6, jnp.asarray(dm)).reshape(240, 352)
    b6rep = jnp.repeat(b6, 11)

    b4r = b4.reshape(1, 1)

    xw1, ones16, counts, starts = _prep(xpad, W1, batchp)
    deg2 = _deg(ones16, src3, dst3)
    dinv, y1 = _scale(deg2, xw1)
    agg1 = _mp32(y1, src3, dst3)
    x1, y2 = _layer32(agg1, y1, dinv, b1, W2)
    agg2 = _mp32(y2, src3, dst3)
    x2, y3 = _layer32(agg2, y2, dinv, b2, W3)
    agg3 = _mp32(y3, src3, dst3)
    x3, y4 = _layer16(agg3, y3, dinv, b3, W4p)
    agg4 = _mp16(y4, src3, dst3)
    xc, keys = _finish(agg4, y4, dinv, b4r, x1, x2, x3)
    pooled = _topk(keys.reshape(NP), starts.reshape(B), counts.reshape(B), xc)
    out = _tail(pooled.reshape(B, 32 * 112), m5, b5rep, se, so, g6, b6rep,
                Wc1, bc1, Wc2, bc2)
    return out
